# Initial kernel scaffold; baseline (speedup 1.0000x reference)
import jax
import jax.numpy as jnp
from jax.experimental import pallas as pl


def kernel(x, layer_id, tables):
    raise NotImplementedError("write your pallas kernel here")


# speedup vs baseline: 2.5422x; 2.5422x over previous
import functools

import jax
import jax.numpy as jnp
from jax import lax
from jax.experimental import pallas as pl
from jax.experimental.pallas import tpu as pltpu
from jax.experimental.pallas import tpu_sc as plsc

_NC = 2
_NS = 16
_LANES = 16
_NW = _NC * _NS


def kernel(x, layer_id, tables):
    n_layers, vocab, d = tables.shape
    b, s = x.shape
    n = b * s
    b_per_w = n // _NW

    flat_tables = tables.reshape(n_layers * vocab, d)
    idx = x.reshape(n)
    off = jnp.full((_LANES,), jnp.int32(layer_id) * vocab, dtype=jnp.int32)

    mesh = plsc.VectorSubcoreMesh(core_axis_name="c", subcore_axis_name="s")

    @functools.partial(
        pl.kernel,
        mesh=mesh,
        out_type=jax.ShapeDtypeStruct((n, d), tables.dtype),
        scratch_types=[
            pltpu.VMEM((b_per_w,), jnp.int32),
            pltpu.VMEM((_LANES,), jnp.int32),
            pltpu.VMEM((b_per_w, d), jnp.float32),
            pltpu.SemaphoreType.DMA,
        ],
    )
    def gather_kernel(table_hbm, idx_hbm, off_hbm, out_hbm,
                      idx_v, off_v, rows_v, sem):
        wid = lax.axis_index("s") * _NC + lax.axis_index("c")
        base = wid * b_per_w
        pltpu.sync_copy(idx_hbm.at[pl.ds(base, b_per_w)], idx_v)
        pltpu.sync_copy(off_hbm, off_v)
        off_reg = off_v[...]

        @pl.loop(0, b_per_w, step=_LANES)
        def _(i):
            slc = pl.ds(i, _LANES)
            idx_v.at[slc][...] = idx_v.at[slc][...] + off_reg

        pltpu.async_copy(table_hbm.at[idx_v], rows_v, sem).wait()
        pltpu.sync_copy(rows_v, out_hbm.at[pl.ds(base, b_per_w)])

    out = gather_kernel(flat_tables, idx, off)
    return out.reshape(b, s, d)
